# Initial kernel scaffold; baseline (speedup 1.0000x reference)
import jax
import jax.numpy as jnp
from jax.experimental import pallas as pl


def kernel(x, edge_index, edge_weight, W1, b1, W2, b2):
    raise NotImplementedError("write your pallas kernel here")


# speedup vs baseline: 5.4229x; 5.4229x over previous
import functools

import jax
import jax.numpy as jnp
from jax import lax
from jax.experimental import pallas as pl
from jax.experimental.pallas import tpu as pltpu
from jax.experimental.pallas import tpu_sc as plsc

N_NODES = 10000
N_EDGES = 160000
D = 256
HALF = 128

NC = 2
NS = 16

DEG_EPW = N_EDGES // (NC * NS)
DEG_CH = 100
DEG_NCH = DEG_EPW // DEG_CH

STR_OFF = 624
STR_LEN = 640

AGG_EPT = N_EDGES // NS
AGG_CH = 80
AGG_NCH = AGG_EPT // AGG_CH

PASS_ROWS = 5000
ACC_ROWS = 5008
PSTR_OFF = 312
PSTR_LEN = 328

_mesh = functools.partial(
    plsc.VectorSubcoreMesh,
    core_axis_name="c", subcore_axis_name="s",
    num_cores=NC, num_subcores=NS)


def _sc_deg_body(dst_hbm, ew_hbm, out_hbm, dstv, ewv, zb, deg_sh):
    c = lax.axis_index("c")
    s = lax.axis_index("s")
    for i in range(STR_LEN // 16):
        zb[pl.ds(i * 16, 16)] = jnp.zeros((16,), jnp.float32)
    pltpu.sync_copy(zb, deg_sh.at[pl.ds(s * STR_OFF, STR_LEN)])
    pltpu.sync_copy(dst_hbm.at[c, s], dstv)
    pltpu.sync_copy(ew_hbm.at[c, s], ewv)
    plsc.subcore_barrier()

    def add_chunk(k, carry):
        pltpu.sync_copy(ewv.at[k], deg_sh.at[dstv.at[k]], add=True)
        return carry

    lax.fori_loop(0, DEG_NCH, add_chunk, 0)
    plsc.subcore_barrier()
    pltpu.sync_copy(deg_sh.at[pl.ds(s * STR_OFF, STR_LEN)], zb)
    pltpu.sync_copy(zb, out_hbm.at[pl.ds(c * N_NODES + s * STR_OFF, STR_LEN)])


@jax.jit
def _sc_deg(dst4, ew4):
    return pl.kernel(
        _sc_deg_body,
        out_type=jax.ShapeDtypeStruct((NC * N_NODES,), jnp.float32),
        mesh=_mesh(),
        scratch_types=[
            pltpu.VMEM((DEG_NCH, DEG_CH), jnp.int32),
            pltpu.VMEM((DEG_NCH, DEG_CH), jnp.float32),
            pltpu.VMEM((STR_LEN,), jnp.float32),
            pltpu.VMEM_SHARED((N_NODES,), jnp.float32),
        ],
    )(dst4, ew4)


def _sc_agg_body(p, src_hbm, dst_hbm, ew_hbm, hp0_hbm, hp1_hbm,
                 out0_hbm, out1_hbm,
                 srcv, dstv, ewv, buf, acc_sh, sem):
    c = lax.axis_index("c")
    s = lax.axis_index("s")

    def zero_row(i, carry):
        for j in range(HALF // 16):
            buf[i, pl.ds(j * 16, 16)] = jnp.zeros((16,), jnp.float32)
        return carry

    lax.fori_loop(0, AGG_CH, zero_row, 0)

    for off, ln in ((0, 80), (80, 80), (160, 80), (240, 80), (320, 8)):
        pltpu.sync_copy(buf.at[pl.ds(0, ln)],
                        acc_sh.at[pl.ds(s * PSTR_OFF + off, ln)])

    pltpu.sync_copy(src_hbm.at[s], srcv)
    pltpu.sync_copy(dst_hbm.at[s], dstv)
    pltpu.sync_copy(ew_hbm.at[s], ewv)

    slack = PASS_ROWS + lax.rem(s, 8)

    def adjust(k, carry):
        for g in range(AGG_CH // 16):
            sl = pl.ds(g * 16, 16)
            ld = dstv[k, sl] - p * PASS_ROWS
            inr = (ld >= 0) & (ld < PASS_ROWS)
            dstv[k, sl] = jnp.where(inr, ld, slack)
            ewv[k, sl] = jnp.where(inr, ewv[k, sl], 0.0)
        return carry

    lax.fori_loop(0, AGG_NCH, adjust, 0)
    plsc.subcore_barrier()

    def chunk(k, carry):
        @pl.when(c == 0)
        def _():
            pltpu.async_copy(hp0_hbm.at[srcv.at[k]], buf, sem).wait()

        @pl.when(c == 1)
        def _():
            pltpu.async_copy(hp1_hbm.at[srcv.at[k]], buf, sem).wait()

        def group(g, gcarry):
            w16 = ewv[k, pl.ds(g * 16, 16)]
            for l in range(16):
                w = jnp.broadcast_to(w16[l:l + 1], (16,))
                row = g * 16 + l
                for j in range(HALF // 16):
                    buf[row, pl.ds(j * 16, 16)] = buf[row, pl.ds(j * 16, 16)] * w
            return gcarry

        lax.fori_loop(0, AGG_CH // 16, group, 0)
        pltpu.sync_copy(buf, acc_sh.at[dstv.at[k]], add=True)
        return carry

    lax.fori_loop(0, AGG_NCH, chunk, 0)
    plsc.subcore_barrier()

    def wb_one(off, ln):
        pltpu.sync_copy(acc_sh.at[pl.ds(off, ln)], buf.at[pl.ds(0, ln)])

        @pl.when(c == 0)
        def _():
            pltpu.sync_copy(buf.at[pl.ds(0, ln)], out0_hbm.at[pl.ds(off, ln)])

        @pl.when(c == 1)
        def _():
            pltpu.sync_copy(buf.at[pl.ds(0, ln)], out1_hbm.at[pl.ds(off, ln)])

    for off, ln in ((0, 80), (80, 80), (160, 80), (240, 80), (320, 8)):
        wb_one(s * PSTR_OFF + off, ln)


def _make_agg(p):
    @jax.jit
    def agg(src3, dst3, ew3, hp0, hp1):
        return pl.kernel(
            functools.partial(_sc_agg_body, p),
            out_type=[jax.ShapeDtypeStruct((ACC_ROWS, HALF), jnp.float32),
                      jax.ShapeDtypeStruct((ACC_ROWS, HALF), jnp.float32)],
            mesh=_mesh(),
            scratch_types=[
                pltpu.VMEM((AGG_NCH, AGG_CH), jnp.int32),
                pltpu.VMEM((AGG_NCH, AGG_CH), jnp.int32),
                pltpu.VMEM((AGG_NCH, AGG_CH), jnp.float32),
                pltpu.VMEM((AGG_CH, HALF), jnp.float32),
                pltpu.VMEM_SHARED((ACC_ROWS, HALF), jnp.float32),
                pltpu.SemaphoreType.DMA,
            ],
        )(src3, dst3, ew3, hp0, hp1)

    return agg


_sc_agg_p0 = _make_agg(0)
_sc_agg_p1 = _make_agg(1)


ROWS_BLK = 2000
GRID = N_NODES // ROWS_BLK


def _dinv_of(deg_ref):
    deg = deg_ref[0, 0, :] + deg_ref[0, 1, :] + 1.0
    return lax.rsqrt(deg)


def _h_blk():
    return pl.BlockSpec((ROWS_BLK, HALF), lambda i: (i, 0))


def _deg_spec():
    return pl.BlockSpec((1, NC, ROWS_BLK), lambda i: (i, 0, 0))


def _h_shapes():
    return [jax.ShapeDtypeStruct((N_NODES, HALF), jnp.float32)] * 2


def _tc_a_body(x_ref, w_ref, deg_ref, o0, o1):
    h = jnp.dot(x_ref[...], w_ref[...], preferred_element_type=jnp.float32)
    hp = h * _dinv_of(deg_ref)[:, None]
    o0[...] = hp[:, :HALF]
    o1[...] = hp[:, HALF:]


@jax.jit
def _tc_a(x, W1, degr):
    return pl.pallas_call(
        _tc_a_body,
        grid=(GRID,),
        in_specs=[
            pl.BlockSpec((ROWS_BLK, D), lambda i: (i, 0)),
            pl.BlockSpec((D, D), lambda i: (0, 0)),
            _deg_spec(),
        ],
        out_specs=[_h_blk(), _h_blk()],
        out_shape=_h_shapes(),
    )(x, W1, degr)


def _tc_b_body(a0, a1, h0, h1, deg_ref, b_ref, w_ref, o0, o1):
    dinv = _dinv_of(deg_ref)[:, None]
    b = b_ref[...]
    u_lo = dinv * (a0[...] + h0[...]) + b[:HALF][None, :]
    u_hi = dinv * (a1[...] + h1[...]) + b[HALF:][None, :]
    z = jnp.maximum(jnp.concatenate([u_lo, u_hi], axis=1), 0.0)
    h2 = jnp.dot(z, w_ref[...], preferred_element_type=jnp.float32) * dinv
    o0[...] = h2[:, :HALF]
    o1[...] = h2[:, HALF:]


@jax.jit
def _tc_b(a0, a1, h0, h1, degr, b1, W2):
    return pl.pallas_call(
        _tc_b_body,
        grid=(GRID,),
        in_specs=[_h_blk(), _h_blk(), _h_blk(), _h_blk(),
                  _deg_spec(),
                  pl.BlockSpec((D,), lambda i: (0,)),
                  pl.BlockSpec((D, D), lambda i: (0, 0))],
        out_specs=[_h_blk(), _h_blk()],
        out_shape=_h_shapes(),
    )(a0, a1, h0, h1, degr, b1, W2)


def _tc_c_body(a0, a1, h0, h1, deg_ref, b_ref, out_ref):
    dinv = _dinv_of(deg_ref)[:, None]
    lo = dinv * (a0[...] + h0[...])
    hi = dinv * (a1[...] + h1[...])
    out_ref[...] = jnp.concatenate([lo, hi], axis=1) + b_ref[...][None, :]


@jax.jit
def _tc_c(a0, a1, h0, h1, degr, b2):
    return pl.pallas_call(
        _tc_c_body,
        grid=(GRID,),
        in_specs=[_h_blk(), _h_blk(), _h_blk(), _h_blk(),
                  _deg_spec(),
                  pl.BlockSpec((D,), lambda i: (0,))],
        out_specs=pl.BlockSpec((ROWS_BLK, D), lambda i: (i, 0)),
        out_shape=jax.ShapeDtypeStruct((N_NODES, D), jnp.float32),
    )(a0, a1, h0, h1, degr, b2)


def _agg_both(src3, dst3, ew3, hp0, hp1):
    aA0, aA1 = _sc_agg_p0(src3, dst3, ew3, hp0, hp1)
    aB0, aB1 = _sc_agg_p1(src3, dst3, ew3, hp0, hp1)
    a0 = jnp.concatenate([aA0[:PASS_ROWS], aB0[:PASS_ROWS]], axis=0)
    a1 = jnp.concatenate([aA1[:PASS_ROWS], aB1[:PASS_ROWS]], axis=0)
    return a0, a1


def kernel(x, edge_index, edge_weight, W1, b1, W2, b2):
    src = edge_index[0].astype(jnp.int32)
    dst = edge_index[1].astype(jnp.int32)
    ew = edge_weight.astype(jnp.float32)

    dst4 = dst.reshape(NC, NS, DEG_NCH, DEG_CH)
    ew4 = ew.reshape(NC, NS, DEG_NCH, DEG_CH)
    src3 = src.reshape(NS, AGG_NCH, AGG_CH)
    dst3 = dst.reshape(NS, AGG_NCH, AGG_CH)
    ew3 = ew.reshape(NS, AGG_NCH, AGG_CH)

    degf = _sc_deg(dst4, ew4)
    degr = degf.reshape(NC, GRID, ROWS_BLK).transpose(1, 0, 2)

    h10, h11 = _tc_a(x, W1, degr)
    a10, a11 = _agg_both(src3, dst3, ew3, h10, h11)
    h20, h21 = _tc_b(a10, a11, h10, h11, degr, b1, W2)
    a20, a21 = _agg_both(src3, dst3, ew3, h20, h21)
    return _tc_c(a20, a21, h20, h21, degr, b2)


# speedup vs baseline: 9.1878x; 1.6943x over previous
import functools

import jax
import jax.numpy as jnp
from jax import lax
from jax.experimental import pallas as pl
from jax.experimental.pallas import tpu as pltpu
from jax.experimental.pallas import tpu_sc as plsc

N_NODES = 10000
N_EDGES = 160000
D = 256
HALF = 128

NC = 2
NS = 16

DEG_EPW = N_EDGES // (NC * NS)
DEG_CH = 100
DEG_NCH = DEG_EPW // DEG_CH

STR_OFF = 624
STR_LEN = 640

AGG_EPT = N_EDGES // NS
AGG_CH = 80
AGG_NCH = AGG_EPT // AGG_CH

PASS_ROWS = 5000
ACC_ROWS = 5008
PSTR_OFF = 312
PSTR_LEN = 328

_mesh = functools.partial(
    plsc.VectorSubcoreMesh,
    core_axis_name="c", subcore_axis_name="s",
    num_cores=NC, num_subcores=NS)


def _sc_deg_body(dst_hbm, ew_hbm, out_hbm, dstv, ewv, zb, deg_sh):
    c = lax.axis_index("c")
    s = lax.axis_index("s")
    for i in range(STR_LEN // 16):
        zb[pl.ds(i * 16, 16)] = jnp.zeros((16,), jnp.float32)
    pltpu.sync_copy(zb, deg_sh.at[pl.ds(s * STR_OFF, STR_LEN)])
    pltpu.sync_copy(dst_hbm.at[c, s], dstv)
    pltpu.sync_copy(ew_hbm.at[c, s], ewv)
    plsc.subcore_barrier()

    def add_chunk(k, carry):
        pltpu.sync_copy(ewv.at[k], deg_sh.at[dstv.at[k]], add=True)
        return carry

    lax.fori_loop(0, DEG_NCH, add_chunk, 0)
    plsc.subcore_barrier()
    pltpu.sync_copy(deg_sh.at[pl.ds(s * STR_OFF, STR_LEN)], zb)
    pltpu.sync_copy(zb, out_hbm.at[pl.ds(c * N_NODES + s * STR_OFF, STR_LEN)])


@jax.jit
def _sc_deg(dst4, ew4):
    return pl.kernel(
        _sc_deg_body,
        out_type=jax.ShapeDtypeStruct((NC * N_NODES,), jnp.float32),
        mesh=_mesh(),
        scratch_types=[
            pltpu.VMEM((DEG_NCH, DEG_CH), jnp.int32),
            pltpu.VMEM((DEG_NCH, DEG_CH), jnp.float32),
            pltpu.VMEM((STR_LEN,), jnp.float32),
            pltpu.VMEM_SHARED((N_NODES,), jnp.float32),
        ],
    )(dst4, ew4)


def _sc_agg_body(p, src_hbm, dst_hbm, ew_hbm, hp0_hbm, hp1_hbm,
                 out0_hbm, out1_hbm,
                 srcv, dstv, ewv, buf0, buf1, acc_sh, sem0, sem1):
    c = lax.axis_index("c")
    s = lax.axis_index("s")

    def zero_row(i, carry):
        for j in range(HALF // 16):
            buf0[i, pl.ds(j * 16, 16)] = jnp.zeros((16,), jnp.float32)
        return carry

    lax.fori_loop(0, AGG_CH, zero_row, 0)

    for off, ln in ((0, 80), (80, 80), (160, 80), (240, 80), (320, 8)):
        pltpu.sync_copy(buf0.at[pl.ds(0, ln)],
                        acc_sh.at[pl.ds(s * PSTR_OFF + off, ln)])

    pltpu.sync_copy(src_hbm.at[s], srcv)
    pltpu.sync_copy(dst_hbm.at[s], dstv)
    pltpu.sync_copy(ew_hbm.at[s], ewv)

    slack = PASS_ROWS + lax.rem(s, 8)

    def adjust(k, carry):
        for g in range(AGG_CH // 16):
            sl = pl.ds(g * 16, 16)
            ld = dstv[k, sl] - p * PASS_ROWS
            inr = (ld >= 0) & (ld < PASS_ROWS)
            dstv[k, sl] = jnp.where(inr, ld, slack)
            ewv[k, sl] = jnp.where(inr, ewv[k, sl], 0.0)
        return carry

    lax.fori_loop(0, AGG_NCH, adjust, 0)
    plsc.subcore_barrier()

    def start_gather(k, buf, sem):
        @pl.when(c == 0)
        def _():
            pltpu.async_copy(hp0_hbm.at[srcv.at[k]], buf, sem)

        @pl.when(c == 1)
        def _():
            pltpu.async_copy(hp1_hbm.at[srcv.at[k]], buf, sem)

    def wait_gather(buf, sem):
        pltpu.make_async_copy(hp0_hbm.at[pl.ds(0, AGG_CH)], buf, sem).wait()

    def process(k, buf):
        def group(g, gcarry):
            w16 = ewv[k, pl.ds(g * 16, 16)]
            for l in range(16):
                w = jnp.broadcast_to(w16[l:l + 1], (16,))
                row = g * 16 + l
                for j in range(HALF // 16):
                    buf[row, pl.ds(j * 16, 16)] = buf[row, pl.ds(j * 16, 16)] * w
            return gcarry

        lax.fori_loop(0, AGG_CH // 16, group, 0)
        pltpu.sync_copy(buf, acc_sh.at[dstv.at[k]], add=True)

    start_gather(0, buf0, sem0)

    def pair(g, carry):
        k0 = 2 * g
        start_gather(k0 + 1, buf1, sem1)
        wait_gather(buf0, sem0)
        process(k0, buf0)
        start_gather(k0 + 2, buf0, sem0)
        wait_gather(buf1, sem1)
        process(k0 + 1, buf1)
        return carry

    lax.fori_loop(0, AGG_NCH // 2, pair, 0)
    wait_gather(buf0, sem0)
    process(AGG_NCH - 1, buf0)
    plsc.subcore_barrier()

    def wb_one(off, ln):
        pltpu.sync_copy(acc_sh.at[pl.ds(off, ln)], buf0.at[pl.ds(0, ln)])

        @pl.when(c == 0)
        def _():
            pltpu.sync_copy(buf0.at[pl.ds(0, ln)], out0_hbm.at[pl.ds(off, ln)])

        @pl.when(c == 1)
        def _():
            pltpu.sync_copy(buf0.at[pl.ds(0, ln)], out1_hbm.at[pl.ds(off, ln)])

    for off, ln in ((0, 80), (80, 80), (160, 80), (240, 80), (320, 8)):
        wb_one(s * PSTR_OFF + off, ln)


def _make_agg(p):
    @jax.jit
    def agg(src3, dst3, ew3, hp0, hp1):
        return pl.kernel(
            functools.partial(_sc_agg_body, p),
            out_type=[jax.ShapeDtypeStruct((ACC_ROWS, HALF), jnp.float32),
                      jax.ShapeDtypeStruct((ACC_ROWS, HALF), jnp.float32)],
            mesh=_mesh(),
            scratch_types=[
                pltpu.VMEM((AGG_NCH, AGG_CH), jnp.int32),
                pltpu.VMEM((AGG_NCH, AGG_CH), jnp.int32),
                pltpu.VMEM((AGG_NCH, AGG_CH), jnp.float32),
                pltpu.VMEM((AGG_CH, HALF), jnp.float32),
                pltpu.VMEM((AGG_CH, HALF), jnp.float32),
                pltpu.VMEM_SHARED((ACC_ROWS, HALF), jnp.float32),
                pltpu.SemaphoreType.DMA,
                pltpu.SemaphoreType.DMA,
            ],
        )(src3, dst3, ew3, hp0, hp1)

    return agg


_sc_agg_p0 = _make_agg(0)
_sc_agg_p1 = _make_agg(1)


ROWS_BLK = 2000
GRID = N_NODES // ROWS_BLK


def _dinv_of(deg_ref):
    deg = deg_ref[0, 0, :] + deg_ref[0, 1, :] + 1.0
    return lax.rsqrt(deg)


def _h_blk():
    return pl.BlockSpec((ROWS_BLK, HALF), lambda i: (i, 0))


def _deg_spec():
    return pl.BlockSpec((1, NC, ROWS_BLK), lambda i: (i, 0, 0))


def _h_shapes():
    return [jax.ShapeDtypeStruct((N_NODES, HALF), jnp.float32)] * 2


def _tc_a_body(x_ref, w_ref, deg_ref, o0, o1):
    h = jnp.dot(x_ref[...], w_ref[...], preferred_element_type=jnp.float32)
    hp = h * _dinv_of(deg_ref)[:, None]
    o0[...] = hp[:, :HALF]
    o1[...] = hp[:, HALF:]


@jax.jit
def _tc_a(x, W1, degr):
    return pl.pallas_call(
        _tc_a_body,
        grid=(GRID,),
        in_specs=[
            pl.BlockSpec((ROWS_BLK, D), lambda i: (i, 0)),
            pl.BlockSpec((D, D), lambda i: (0, 0)),
            _deg_spec(),
        ],
        out_specs=[_h_blk(), _h_blk()],
        out_shape=_h_shapes(),
    )(x, W1, degr)


def _tc_b_body(a0, a1, h0, h1, deg_ref, b_ref, w_ref, o0, o1):
    dinv = _dinv_of(deg_ref)[:, None]
    b = b_ref[...]
    u_lo = dinv * (a0[...] + h0[...]) + b[:HALF][None, :]
    u_hi = dinv * (a1[...] + h1[...]) + b[HALF:][None, :]
    z = jnp.maximum(jnp.concatenate([u_lo, u_hi], axis=1), 0.0)
    h2 = jnp.dot(z, w_ref[...], preferred_element_type=jnp.float32) * dinv
    o0[...] = h2[:, :HALF]
    o1[...] = h2[:, HALF:]


@jax.jit
def _tc_b(a0, a1, h0, h1, degr, b1, W2):
    return pl.pallas_call(
        _tc_b_body,
        grid=(GRID,),
        in_specs=[_h_blk(), _h_blk(), _h_blk(), _h_blk(),
                  _deg_spec(),
                  pl.BlockSpec((D,), lambda i: (0,)),
                  pl.BlockSpec((D, D), lambda i: (0, 0))],
        out_specs=[_h_blk(), _h_blk()],
        out_shape=_h_shapes(),
    )(a0, a1, h0, h1, degr, b1, W2)


def _tc_c_body(a0, a1, h0, h1, deg_ref, b_ref, out_ref):
    dinv = _dinv_of(deg_ref)[:, None]
    lo = dinv * (a0[...] + h0[...])
    hi = dinv * (a1[...] + h1[...])
    out_ref[...] = jnp.concatenate([lo, hi], axis=1) + b_ref[...][None, :]


@jax.jit
def _tc_c(a0, a1, h0, h1, degr, b2):
    return pl.pallas_call(
        _tc_c_body,
        grid=(GRID,),
        in_specs=[_h_blk(), _h_blk(), _h_blk(), _h_blk(),
                  _deg_spec(),
                  pl.BlockSpec((D,), lambda i: (0,))],
        out_specs=pl.BlockSpec((ROWS_BLK, D), lambda i: (i, 0)),
        out_shape=jax.ShapeDtypeStruct((N_NODES, D), jnp.float32),
    )(a0, a1, h0, h1, degr, b2)


def _agg_both(src3, dst3, ew3, hp0, hp1):
    aA0, aA1 = _sc_agg_p0(src3, dst3, ew3, hp0, hp1)
    aB0, aB1 = _sc_agg_p1(src3, dst3, ew3, hp0, hp1)
    a0 = jnp.concatenate([aA0[:PASS_ROWS], aB0[:PASS_ROWS]], axis=0)
    a1 = jnp.concatenate([aA1[:PASS_ROWS], aB1[:PASS_ROWS]], axis=0)
    return a0, a1


def kernel(x, edge_index, edge_weight, W1, b1, W2, b2):
    src = edge_index[0].astype(jnp.int32)
    dst = edge_index[1].astype(jnp.int32)
    ew = edge_weight.astype(jnp.float32)

    dst4 = dst.reshape(NC, NS, DEG_NCH, DEG_CH)
    ew4 = ew.reshape(NC, NS, DEG_NCH, DEG_CH)
    src3 = src.reshape(NS, AGG_NCH, AGG_CH)
    dst3 = dst.reshape(NS, AGG_NCH, AGG_CH)
    ew3 = ew.reshape(NS, AGG_NCH, AGG_CH)

    degf = _sc_deg(dst4, ew4)
    degr = degf.reshape(NC, GRID, ROWS_BLK).transpose(1, 0, 2)

    h10, h11 = _tc_a(x, W1, degr)
    a10, a11 = _agg_both(src3, dst3, ew3, h10, h11)
    h20, h21 = _tc_b(a10, a11, h10, h11, degr, b1, W2)
    a20, a21 = _agg_both(src3, dst3, ew3, h20, h21)
    return _tc_c(a20, a21, h20, h21, degr, b2)


# speedup vs baseline: 10.4091x; 1.1329x over previous
import functools

import jax
import jax.numpy as jnp
from jax import lax
from jax.experimental import pallas as pl
from jax.experimental.pallas import tpu as pltpu
from jax.experimental.pallas import tpu_sc as plsc

N_NODES = 10000
N_EDGES = 160000
D = 256
HALF = 128

NC = 2
NS = 16

DEG_EPW = N_EDGES // (NC * NS)
DEG_CH = 100
DEG_NCH = DEG_EPW // DEG_CH

STR_OFF = 624
STR_LEN = 640

AGG_EPT = N_EDGES // NS
AGG_CH = 80
AGG_NCH = AGG_EPT // AGG_CH

PASS_ROWS = 5000
ACC_ROWS = 5008
PSTR_OFF = 312
PSTR_LEN = 328

_mesh = functools.partial(
    plsc.VectorSubcoreMesh,
    core_axis_name="c", subcore_axis_name="s",
    num_cores=NC, num_subcores=NS)


def _sc_deg_body(dst_hbm, ew_hbm, out_hbm, dstv, ewv, zb, deg_sh):
    c = lax.axis_index("c")
    s = lax.axis_index("s")
    for i in range(STR_LEN // 16):
        zb[pl.ds(i * 16, 16)] = jnp.zeros((16,), jnp.float32)
    pltpu.sync_copy(zb, deg_sh.at[pl.ds(s * STR_OFF, STR_LEN)])
    pltpu.sync_copy(dst_hbm.at[c, s], dstv)
    pltpu.sync_copy(ew_hbm.at[c, s], ewv)
    plsc.subcore_barrier()

    def add_chunk(k, carry):
        pltpu.sync_copy(ewv.at[k], deg_sh.at[dstv.at[k]], add=True)
        return carry

    lax.fori_loop(0, DEG_NCH, add_chunk, 0)
    plsc.subcore_barrier()
    pltpu.sync_copy(deg_sh.at[pl.ds(s * STR_OFF, STR_LEN)], zb)
    pltpu.sync_copy(zb, out_hbm.at[pl.ds(c * N_NODES + s * STR_OFF, STR_LEN)])


@jax.jit
def _sc_deg(dst4, ew4):
    return pl.kernel(
        _sc_deg_body,
        out_type=jax.ShapeDtypeStruct((NC * N_NODES,), jnp.float32),
        mesh=_mesh(),
        scratch_types=[
            pltpu.VMEM((DEG_NCH, DEG_CH), jnp.int32),
            pltpu.VMEM((DEG_NCH, DEG_CH), jnp.float32),
            pltpu.VMEM((STR_LEN,), jnp.float32),
            pltpu.VMEM_SHARED((N_NODES,), jnp.float32),
        ],
    )(dst4, ew4)


def _sc_agg_body(p, src_hbm, dst_hbm, ew_hbm, hp0_hbm, hp1_hbm,
                 out0_hbm, out1_hbm,
                 srcv, dstv, ewv, buf0, buf1, buf2, acc_sh,
                 sem0, sem1, sem2, sems0, sems1, sems2):
    c = lax.axis_index("c")
    s = lax.axis_index("s")

    def zero_row(i, carry):
        for j in range(HALF // 16):
            buf0[i, pl.ds(j * 16, 16)] = jnp.zeros((16,), jnp.float32)
        return carry

    lax.fori_loop(0, AGG_CH, zero_row, 0)

    for off, ln in ((0, 80), (80, 80), (160, 80), (240, 80), (320, 8)):
        pltpu.sync_copy(buf0.at[pl.ds(0, ln)],
                        acc_sh.at[pl.ds(s * PSTR_OFF + off, ln)])

    pltpu.sync_copy(src_hbm.at[s], srcv)
    pltpu.sync_copy(dst_hbm.at[s], dstv)
    pltpu.sync_copy(ew_hbm.at[s], ewv)

    slack = PASS_ROWS + lax.rem(s, 8)

    def adjust(k, carry):
        for g in range(AGG_CH // 16):
            sl = pl.ds(g * 16, 16)
            ld = dstv[k, sl] - p * PASS_ROWS
            inr = (ld >= 0) & (ld < PASS_ROWS)
            dstv[k, sl] = jnp.where(inr, ld, slack)
            ewv[k, sl] = jnp.where(inr, ewv[k, sl], 0.0)
        return carry

    lax.fori_loop(0, AGG_NCH, adjust, 0)
    plsc.subcore_barrier()

    def start_gather(k, buf, sem):
        @pl.when(c == 0)
        def _():
            pltpu.async_copy(hp0_hbm.at[srcv.at[k]], buf, sem)

        @pl.when(c == 1)
        def _():
            pltpu.async_copy(hp1_hbm.at[srcv.at[k]], buf, sem)

    def wait_gather(buf, sem):
        pltpu.make_async_copy(hp0_hbm.at[pl.ds(0, AGG_CH)], buf, sem).wait()

    def start_scatter(k, buf, sem):
        pltpu.async_copy(buf, acc_sh.at[dstv.at[k]], sem, add=True)

    def wait_scatter(buf, sem):
        pltpu.make_async_copy(buf, acc_sh.at[pl.ds(0, AGG_CH)], sem).wait()

    def scale(k, buf):
        def group(g, gcarry):
            w16 = ewv[k, pl.ds(g * 16, 16)]
            for l in range(16):
                w = jnp.broadcast_to(w16[l:l + 1], (16,))
                row = g * 16 + l
                for j in range(HALF // 16):
                    buf[row, pl.ds(j * 16, 16)] = buf[row, pl.ds(j * 16, 16)] * w
            return gcarry

        lax.fori_loop(0, AGG_CH // 16, group, 0)

    bufs = (buf0, buf1, buf2)
    gsems = (sem0, sem1, sem2)
    ssems = (sems0, sems1, sems2)
    start_gather(0, buf0, sem0)
    start_gather(1, buf1, sem1)

    def triple(t, carry):
        for j in range(3):
            kk = 3 * t + j
            b, sg, ss = bufs[j], gsems[j], ssems[j]
            bp, sgp, ssp = bufs[j - 1], gsems[j - 1], ssems[j - 1]

            @pl.when(kk < AGG_NCH)
            def _():
                wait_gather(b, sg)
                scale(kk, b)
                start_scatter(kk, b, ss)

                @pl.when(kk >= 1)
                def _():
                    wait_scatter(bp, ssp)

                @pl.when(kk + 2 < AGG_NCH)
                def _():
                    start_gather(kk + 2, bp, sgp)

        return carry

    lax.fori_loop(0, (AGG_NCH + 2) // 3, triple, 0)
    wait_scatter(bufs[(AGG_NCH - 1) % 3], ssems[(AGG_NCH - 1) % 3])
    plsc.subcore_barrier()

    def wb_one(off, ln):
        pltpu.sync_copy(acc_sh.at[pl.ds(off, ln)], buf0.at[pl.ds(0, ln)])

        @pl.when(c == 0)
        def _():
            pltpu.sync_copy(buf0.at[pl.ds(0, ln)], out0_hbm.at[pl.ds(off, ln)])

        @pl.when(c == 1)
        def _():
            pltpu.sync_copy(buf0.at[pl.ds(0, ln)], out1_hbm.at[pl.ds(off, ln)])

    for off, ln in ((0, 80), (80, 80), (160, 80), (240, 80), (320, 8)):
        wb_one(s * PSTR_OFF + off, ln)


def _make_agg(p):
    @jax.jit
    def agg(src3, dst3, ew3, hp0, hp1):
        return pl.kernel(
            functools.partial(_sc_agg_body, p),
            out_type=[jax.ShapeDtypeStruct((ACC_ROWS, HALF), jnp.float32),
                      jax.ShapeDtypeStruct((ACC_ROWS, HALF), jnp.float32)],
            mesh=_mesh(),
            scratch_types=[
                pltpu.VMEM((AGG_NCH, AGG_CH), jnp.int32),
                pltpu.VMEM((AGG_NCH, AGG_CH), jnp.int32),
                pltpu.VMEM((AGG_NCH, AGG_CH), jnp.float32),
                pltpu.VMEM((AGG_CH, HALF), jnp.float32),
                pltpu.VMEM((AGG_CH, HALF), jnp.float32),
                pltpu.VMEM((AGG_CH, HALF), jnp.float32),
                pltpu.VMEM_SHARED((ACC_ROWS, HALF), jnp.float32),
                pltpu.SemaphoreType.DMA,
                pltpu.SemaphoreType.DMA,
                pltpu.SemaphoreType.DMA,
                pltpu.SemaphoreType.DMA,
                pltpu.SemaphoreType.DMA,
                pltpu.SemaphoreType.DMA,
            ],
        )(src3, dst3, ew3, hp0, hp1)

    return agg


_sc_agg_p0 = _make_agg(0)
_sc_agg_p1 = _make_agg(1)


ROWS_BLK = 2000
GRID = N_NODES // ROWS_BLK


def _dinv_of(deg_ref):
    deg = deg_ref[0, 0, :] + deg_ref[0, 1, :] + 1.0
    return lax.rsqrt(deg)


def _h_blk():
    return pl.BlockSpec((ROWS_BLK, HALF), lambda i: (i, 0))


def _deg_spec():
    return pl.BlockSpec((1, NC, ROWS_BLK), lambda i: (i, 0, 0))


def _h_shapes():
    return [jax.ShapeDtypeStruct((N_NODES, HALF), jnp.float32)] * 2


def _tc_a_body(x_ref, w_ref, deg_ref, o0, o1):
    h = jnp.dot(x_ref[...], w_ref[...], preferred_element_type=jnp.float32)
    hp = h * _dinv_of(deg_ref)[:, None]
    o0[...] = hp[:, :HALF]
    o1[...] = hp[:, HALF:]


@jax.jit
def _tc_a(x, W1, degr):
    return pl.pallas_call(
        _tc_a_body,
        grid=(GRID,),
        in_specs=[
            pl.BlockSpec((ROWS_BLK, D), lambda i: (i, 0)),
            pl.BlockSpec((D, D), lambda i: (0, 0)),
            _deg_spec(),
        ],
        out_specs=[_h_blk(), _h_blk()],
        out_shape=_h_shapes(),
    )(x, W1, degr)


def _tc_b_body(a0, a1, h0, h1, deg_ref, b_ref, w_ref, o0, o1):
    dinv = _dinv_of(deg_ref)[:, None]
    b = b_ref[...]
    u_lo = dinv * (a0[...] + h0[...]) + b[:HALF][None, :]
    u_hi = dinv * (a1[...] + h1[...]) + b[HALF:][None, :]
    z = jnp.maximum(jnp.concatenate([u_lo, u_hi], axis=1), 0.0)
    h2 = jnp.dot(z, w_ref[...], preferred_element_type=jnp.float32) * dinv
    o0[...] = h2[:, :HALF]
    o1[...] = h2[:, HALF:]


@jax.jit
def _tc_b(a0, a1, h0, h1, degr, b1, W2):
    return pl.pallas_call(
        _tc_b_body,
        grid=(GRID,),
        in_specs=[_h_blk(), _h_blk(), _h_blk(), _h_blk(),
                  _deg_spec(),
                  pl.BlockSpec((D,), lambda i: (0,)),
                  pl.BlockSpec((D, D), lambda i: (0, 0))],
        out_specs=[_h_blk(), _h_blk()],
        out_shape=_h_shapes(),
    )(a0, a1, h0, h1, degr, b1, W2)


def _tc_c_body(a0, a1, h0, h1, deg_ref, b_ref, out_ref):
    dinv = _dinv_of(deg_ref)[:, None]
    lo = dinv * (a0[...] + h0[...])
    hi = dinv * (a1[...] + h1[...])
    out_ref[...] = jnp.concatenate([lo, hi], axis=1) + b_ref[...][None, :]


@jax.jit
def _tc_c(a0, a1, h0, h1, degr, b2):
    return pl.pallas_call(
        _tc_c_body,
        grid=(GRID,),
        in_specs=[_h_blk(), _h_blk(), _h_blk(), _h_blk(),
                  _deg_spec(),
                  pl.BlockSpec((D,), lambda i: (0,))],
        out_specs=pl.BlockSpec((ROWS_BLK, D), lambda i: (i, 0)),
        out_shape=jax.ShapeDtypeStruct((N_NODES, D), jnp.float32),
    )(a0, a1, h0, h1, degr, b2)


def _agg_both(src3, dst3, ew3, hp0, hp1):
    aA0, aA1 = _sc_agg_p0(src3, dst3, ew3, hp0, hp1)
    aB0, aB1 = _sc_agg_p1(src3, dst3, ew3, hp0, hp1)
    a0 = jnp.concatenate([aA0[:PASS_ROWS], aB0[:PASS_ROWS]], axis=0)
    a1 = jnp.concatenate([aA1[:PASS_ROWS], aB1[:PASS_ROWS]], axis=0)
    return a0, a1


def kernel(x, edge_index, edge_weight, W1, b1, W2, b2):
    src = edge_index[0].astype(jnp.int32)
    dst = edge_index[1].astype(jnp.int32)
    ew = edge_weight.astype(jnp.float32)

    dst4 = dst.reshape(NC, NS, DEG_NCH, DEG_CH)
    ew4 = ew.reshape(NC, NS, DEG_NCH, DEG_CH)
    src3 = src.reshape(NS, AGG_NCH, AGG_CH)
    dst3 = dst.reshape(NS, AGG_NCH, AGG_CH)
    ew3 = ew.reshape(NS, AGG_NCH, AGG_CH)

    degf = _sc_deg(dst4, ew4)
    degr = degf.reshape(NC, GRID, ROWS_BLK).transpose(1, 0, 2)

    h10, h11 = _tc_a(x, W1, degr)
    a10, a11 = _agg_both(src3, dst3, ew3, h10, h11)
    h20, h21 = _tc_b(a10, a11, h10, h11, degr, b1, W2)
    a20, a21 = _agg_both(src3, dst3, ew3, h20, h21)
    return _tc_c(a20, a21, h20, h21, degr, b2)
